# baseline (speedup 1.0000x reference)
import jax
import jax.numpy as jnp
from jax import lax
from jax.experimental import pallas as pl
from jax.experimental.pallas import tpu as pltpu

_PRE = 128


@jax.jit
def _extract_eos_tc(tokens, mask):
    B, N, D = tokens.shape

    def body(mask_ref, tokens_hbm, out_ref, sem):
        big = jnp.int32(N)
        pre = mask_ref[:, :_PRE]
        iota = lax.broadcasted_iota(jnp.int32, (B, _PRE), 1)
        val = jnp.where(pre != 0, iota, big)
        copies = []
        for b in range(B):
            idx_b = jnp.min(val[b])

            def cond(carry):
                c, found = carry
                return (found >= big) & (c < N // _PRE)

            def fbody(carry):
                c, _ = carry
                ch = mask_ref[pl.ds(b, 1), pl.ds(c * _PRE, _PRE)]
                ciota = lax.broadcasted_iota(jnp.int32, (1, _PRE), 1)
                found = jnp.min(jnp.where(ch != 0, c * _PRE + ciota, big))
                return c + 1, found

            _, idx_b = lax.while_loop(cond, fbody, (jnp.int32(1), idx_b))
            idx_b = jnp.where(idx_b >= big, 0, idx_b)
            cp = pltpu.make_async_copy(
                tokens_hbm.at[b, pl.ds(idx_b, 1), :],
                out_ref.at[pl.ds(b, 1), :],
                sem,
            )
            cp.start()
            copies.append(cp)
        for cp in copies:
            cp.wait()

    return pl.pallas_call(
        body,
        out_shape=jax.ShapeDtypeStruct((B, D), tokens.dtype),
        in_specs=[
            pl.BlockSpec(memory_space=pltpu.VMEM),
            pl.BlockSpec(memory_space=pl.ANY),
        ],
        out_specs=pl.BlockSpec(memory_space=pltpu.VMEM),
        scratch_shapes=[pltpu.SemaphoreType.DMA],
    )(mask, tokens)


def kernel(tokens, eos_token_mask):
    return _extract_eos_tc(tokens, eos_token_mask)


# speedup vs baseline: 1.1502x; 1.1502x over previous
import functools

import jax
import jax.numpy as jnp
from jax import lax
from jax.experimental import pallas as pl
from jax.experimental.pallas import tpu as pltpu


@jax.jit
def _extract_eos_tc(tokens, mask):
    B, N, D = tokens.shape

    def body(mask_ref, tokens_hbm, out_ref, sem):
        m = mask_ref[...]
        iota = lax.broadcasted_iota(jnp.int32, (B, N), 1)
        val = jnp.where(m != 0, iota, jnp.int32(N))
        copies = []
        for b in range(B):
            idx_b = jnp.min(val[b])
            idx_b = jnp.where(idx_b >= N, 0, idx_b)
            cp = pltpu.make_async_copy(
                tokens_hbm.at[b, pl.ds(idx_b, 1), :],
                out_ref.at[pl.ds(b, 1), :],
                sem,
            )
            cp.start()
            copies.append(cp)
        for cp in copies:
            cp.wait()

    return pl.pallas_call(
        body,
        out_shape=jax.ShapeDtypeStruct((B, D), tokens.dtype),
        in_specs=[
            pl.BlockSpec(memory_space=pltpu.VMEM),
            pl.BlockSpec(memory_space=pl.ANY),
        ],
        out_specs=pl.BlockSpec(memory_space=pltpu.VMEM),
        scratch_shapes=[pltpu.SemaphoreType.DMA],
    )(mask, tokens)


def kernel(tokens, eos_token_mask):
    return _extract_eos_tc(tokens, eos_token_mask)


# speedup vs baseline: 1.1699x; 1.0171x over previous
import functools

import jax
import jax.numpy as jnp
from jax import lax
from jax.experimental import pallas as pl
from jax.experimental.pallas import tpu as pltpu


@jax.jit
def _extract_eos_tc(tokens, mask):
    B, N, D = tokens.shape

    def body(mask_ref, tokens_hbm, out_ref, sem):
        m = mask_ref[...]
        iota = lax.broadcasted_iota(jnp.int32, (B, N), 1)
        val = jnp.where(m != 0, iota, jnp.int32(N))
        mins = jnp.min(val, axis=1)
        mins = jnp.where(mins >= N, 0, mins)
        copies = []
        for b in range(B):
            idx_b = mins[b]
            cp = pltpu.make_async_copy(
                tokens_hbm.at[b, pl.ds(idx_b, 1), :],
                out_ref.at[pl.ds(b, 1), :],
                sem,
            )
            cp.start()
            copies.append(cp)
        for cp in copies:
            cp.wait()

    return pl.pallas_call(
        body,
        out_shape=jax.ShapeDtypeStruct((B, D), tokens.dtype),
        in_specs=[
            pl.BlockSpec(memory_space=pltpu.VMEM),
            pl.BlockSpec(memory_space=pl.ANY),
        ],
        out_specs=pl.BlockSpec(memory_space=pltpu.VMEM),
        scratch_shapes=[pltpu.SemaphoreType.DMA],
    )(mask, tokens)


def kernel(tokens, eos_token_mask):
    return _extract_eos_tc(tokens, eos_token_mask)
